# baseline (speedup 1.0000x reference)
import functools

import jax
import jax.numpy as jnp
from jax import lax
from jax.experimental import pallas as pl
from jax.experimental.pallas import tpu as pltpu
from jax.experimental.pallas import tpu_sc as plsc

_MINOR = 128
_NW = 32
_CH = 128


def _sc_body(nr_total, nr_pos, x_hbm, pos_hbm, idx_hbm, out_hbm,
             xbuf, idxbuf, sx0, sx1, sx2, sg0, sg1, sg2, st0, st1, st2):
    cid = lax.axis_index("c")
    sid = lax.axis_index("s")
    wid = sid * 2 + cid
    nr_per_w = nr_total // _NW
    base = pl.multiple_of(wid * nr_per_w, _CH)
    n_chunks = nr_per_w // _CH
    ibase = pl.multiple_of(lax.rem(base, nr_pos) // _CH, 8)

    sem_x = [sx0, sx1, sx2]
    sem_g = [sg0, sg1, sg2]
    sem_st = [st0, st1, st2]

    pltpu.sync_copy(idx_hbm.at[pl.ds(ibase, n_chunks)], idxbuf)

    descs = {}

    def start_load(i):
        descs["x", i] = pltpu.async_copy(
            x_hbm.at[pl.ds(base + i * _CH, _CH)], xbuf.at[i % 3], sem_x[i % 3])

    def start_gather_add(i):
        descs["g", i] = pltpu.async_copy(
            pos_hbm.at[idxbuf.at[i]], xbuf.at[i % 3], sem_g[i % 3], add=True)

    def start_store(i):
        descs["st", i] = pltpu.async_copy(
            xbuf.at[i % 3], out_hbm.at[pl.ds(base + i * _CH, _CH)],
            sem_st[i % 3])

    for i in range(n_chunks + 2):
        if i < n_chunks:
            if i >= 3:
                descs["st", i - 3].wait()
            start_load(i)
        if 0 <= i - 1 < n_chunks:
            descs["x", i - 1].wait()
            start_gather_add(i - 1)
        if 0 <= i - 2 < n_chunks:
            descs["g", i - 2].wait()
            start_store(i - 2)
    for i in range(max(0, n_chunks - 3), n_chunks):
        descs["st", i].wait()


def kernel(x, pos_table):
    B, L, D = x.shape
    nr_total = B * L * D // _MINOR
    nr_pos = L * D // _MINOR
    x2 = x.reshape(nr_total, _MINOR)
    pos2 = pos_table[:L].reshape(nr_pos, _MINOR)
    idx2 = jnp.arange(nr_pos, dtype=jnp.int32).reshape(nr_pos // _MINOR, _MINOR)

    mesh = plsc.VectorSubcoreMesh(core_axis_name="c", subcore_axis_name="s")
    n_chunks = (nr_total // _NW) // _CH
    sc = pl.kernel(
        functools.partial(_sc_body, nr_total, nr_pos),
        out_type=jax.ShapeDtypeStruct((nr_total, _MINOR), jnp.float32),
        mesh=mesh,
        scratch_types=[
            pltpu.VMEM((3, _CH, _MINOR), jnp.float32),
            pltpu.VMEM((n_chunks, _MINOR), jnp.int32),
            pltpu.SemaphoreType.DMA,
            pltpu.SemaphoreType.DMA,
            pltpu.SemaphoreType.DMA,
            pltpu.SemaphoreType.DMA,
            pltpu.SemaphoreType.DMA,
            pltpu.SemaphoreType.DMA,
            pltpu.SemaphoreType.DMA,
            pltpu.SemaphoreType.DMA,
            pltpu.SemaphoreType.DMA,
        ],
    )
    out = sc(x2, pos2, idx2)
    return out.reshape(B, L, D)
